# baseline (speedup 1.0000x reference)
import jax
import jax.numpy as jnp
from jax.experimental import pallas as pl
from jax.experimental.pallas import tpu as pltpu

BATCH = 64
CIN = 128
C = 256
V = 256
K = 16

GROUP = 1


def _s_kernel(x_ref, w_ref, b_ref, a1_ref, a2_ref, out_ref):
    iota_w0 = jax.lax.broadcasted_iota(jnp.int32, (V, V), 0)
    rev_iota = (jnp.float32(V) - iota_w0.astype(jnp.float32))
    bsel = (iota_w0 // 16
            == jax.lax.rem(jax.lax.broadcasted_iota(jnp.int32, (V, V), 1), 16)
            ).astype(jnp.float32)
    jota = jax.lax.broadcasted_iota(jnp.int32, (K, C), 0)
    wmod = jax.lax.rem(jax.lax.broadcasted_iota(jnp.int32, (K, C), 1), 16)
    neg_inf = jnp.float32(-jnp.inf)
    for gi in range(GROUP):
        x_b = x_ref[gi]
        pre = jnp.dot(w_ref[...], x_b, preferred_element_type=jnp.float32)
        pre = pre + b_ref[...]
        g = jax.lax.dot_general(pre, pre, (((0,), (0,)), ((), ())),
                                preferred_element_type=jnp.float32)
        d = jnp.sum(pre * pre, axis=0, keepdims=True)
        inner = -2.0 * jnp.transpose(g)
        p = (-jnp.transpose(d) - inner) - d
        t1 = jnp.dot(a1_ref[...], pre, preferred_element_type=jnp.float32)
        t2 = jnp.dot(a2_ref[...], pre, preferred_element_type=jnp.float32)
        t2_bcast = jnp.broadcast_to(jnp.transpose(t2), (V, V))
        u = jnp.where(wmod == jota, jnp.broadcast_to(t1, (K, C)), 0.0)
        t1p = jnp.dot(u, bsel, preferred_element_type=jnp.float32,
                      precision=jax.lax.Precision.HIGHEST)
        for j in range(K):
            m = jnp.max(p, axis=0, keepdims=True)
            z = jnp.where(p == m, rev_iota, 0.0)
            mx = jnp.max(z, axis=0, keepdims=True)
            onehot = z == mx
            s_row = jnp.max(jnp.where(onehot, t2_bcast, neg_inf),
                            axis=0, keepdims=True)
            out_ref[gi, j:j + 1, :] = s_row + t1p[j:j + 1, :]
            if j + 1 < K:
                p = jnp.where(onehot, neg_inf, p)


def _softmax0_kernel(s_ref, out_ref):
    s = s_ref[...]
    m = jnp.max(s, axis=0, keepdims=True)
    e = jnp.exp(s - m)
    out_ref[...] = e / jnp.sum(e, axis=0, keepdims=True)


def kernel(x, W, b, a):
    a1 = a[:C, 0].reshape(1, C)
    a2 = a[C:, 0].reshape(1, C)
    b2 = b.reshape(C, 1)
    s = pl.pallas_call(
        _s_kernel,
        grid=(BATCH // GROUP,),
        in_specs=[
            pl.BlockSpec((GROUP, CIN, V), lambda i: (i, 0, 0)),
            pl.BlockSpec((C, CIN), lambda i: (0, 0)),
            pl.BlockSpec((C, 1), lambda i: (0, 0)),
            pl.BlockSpec((1, C), lambda i: (0, 0)),
            pl.BlockSpec((1, C), lambda i: (0, 0)),
        ],
        out_specs=pl.BlockSpec((GROUP, K, V), lambda i: (i, 0, 0)),
        out_shape=jax.ShapeDtypeStruct((BATCH, K, V), jnp.float32),
    )(x, W, b2, a1, a2)
    h = pl.pallas_call(
        _softmax0_kernel,
        out_shape=jax.ShapeDtypeStruct((BATCH, K * V), jnp.float32),
    )(s.reshape(BATCH, K * V))
    return jnp.transpose(h.reshape(BATCH, K, V), (0, 2, 1))


# speedup vs baseline: 1.3970x; 1.3970x over previous
import jax
import jax.numpy as jnp
from jax import lax
from jax.experimental import pallas as pl
from jax.experimental.pallas import tpu as pltpu
from jax.experimental.pallas import tpu_sc as plsc

BATCH = 64
CIN = 128
C = 256
V = 256
K = 16

NSPLIT = 2
HALF = BATCH // NSPLIT
NSUB = 32
SLAB = 128


def _pairwise_kernel(x_ref, w_ref, b_ref, a1_ref, a2_ref,
                     p_ref, t1_ref, t2_ref):
    x_b = x_ref[0]
    pre = jnp.dot(w_ref[...], x_b, preferred_element_type=jnp.float32)
    pre = pre + b_ref[...]
    g = jax.lax.dot_general(pre, pre, (((0,), (0,)), ((), ())),
                            preferred_element_type=jnp.float32)
    d = jnp.sum(pre * pre, axis=0, keepdims=True)
    inner = -2.0 * g
    p_ref[0] = (-d - inner) - jnp.transpose(d)
    t1_ref[0] = jnp.dot(a1_ref[...], pre, preferred_element_type=jnp.float32)
    t2_ref[0] = jnp.dot(a2_ref[...], pre, preferred_element_type=jnp.float32)


def _merge16(ak, av, bk, bv):
    rbk = lax.rev(bk, (0,))
    rbv = lax.rev(bv, (0,))
    cm = ak >= rbk
    mk = jnp.where(cm, ak, rbk)
    mv = jnp.where(cm, av, rbv)
    return plsc.sort_key_val(mk, mv, descending=True)


def _make_sc_topk(rows):
    rpw = rows // NSUB
    nslab = rpw // SLAB

    def _sc_topk_body(p_hbm, t2_hbm, out_hbm, slab_buf, t2_buf, out_buf):
        wid = lax.axis_index("s") * 2 + lax.axis_index("c")
        base_row = wid * rpw
        pltpu.sync_copy(t2_hbm.at[pl.ds(base_row, rpw)], t2_buf)
        iota16 = lax.broadcasted_iota(jnp.int32, (16,), 0)
        for slab in range(nslab):
            row0 = base_row + slab * SLAB
            pltpu.sync_copy(p_hbm.at[pl.ds(row0, SLAB)], slab_buf)

            @plsc.parallel_loop(0, SLAB, unroll=3)
            def row_body(r, slab=slab):
                lists = []
                for c in range(V // 16):
                    vals = slab_buf[r, pl.ds(16 * c, 16)]
                    kk, vv = plsc.sort_key_val(vals, iota16 + 16 * c,
                                               descending=True)
                    lists.append((kk, vv))
                while len(lists) > 1:
                    nxt = []
                    for i in range(0, len(lists), 2):
                        ak, av = lists[i]
                        bk, bv = lists[i + 1]
                        nxt.append(_merge16(ak, av, bk, bv))
                    lists = nxt
                top_idx = lists[0][1]
                rsub = slab * SLAB + r
                adj = top_idx + jnp.bitwise_and(rsub, 256)
                tvals = plsc.load_gather(t2_buf, [adj])
                out_buf[r, :] = tvals

            pltpu.sync_copy(out_buf, out_hbm.at[pl.ds(row0, SLAB)])

    return pl.kernel(
        _sc_topk_body,
        out_type=jax.ShapeDtypeStruct((rows, K), jnp.float32),
        mesh=plsc.VectorSubcoreMesh(core_axis_name="c", subcore_axis_name="s"),
        scratch_types=[
            pltpu.VMEM((SLAB, V), jnp.float32),
            pltpu.VMEM((rpw,), jnp.float32),
            pltpu.VMEM((SLAB, K), jnp.float32),
        ],
        compiler_params=pltpu.CompilerParams(needs_layout_passes=False),
    )


def _softmax0_kernel(*refs):
    s_refs = refs[:NSPLIT]
    t_refs = refs[NSPLIT:2 * NSPLIT]
    out_ref = refs[-1]
    parts = []
    for s_ref, t_ref in zip(s_refs, t_refs):
        parts.append(s_ref[...] + jnp.concatenate([t_ref[...]] * K, axis=1))
    s = jnp.concatenate(parts, axis=0)
    m = jnp.max(s, axis=0, keepdims=True)
    e = jnp.exp(s - m)
    out_ref[...] = e / jnp.sum(e, axis=0, keepdims=True)


def kernel(x, W, b, a):
    a1 = a[:C, 0].reshape(1, C)
    a2 = a[C:, 0].reshape(1, C)
    b2 = b.reshape(C, 1)
    sc_topk = _make_sc_topk(HALF * V)

    def half(x_half):
        rows = HALF * V
        p, t1, t2 = pl.pallas_call(
            _pairwise_kernel,
            grid=(HALF,),
            in_specs=[
                pl.BlockSpec((1, CIN, V), lambda i: (i, 0, 0)),
                pl.BlockSpec((C, CIN), lambda i: (0, 0)),
                pl.BlockSpec((C, 1), lambda i: (0, 0)),
                pl.BlockSpec((1, C), lambda i: (0, 0)),
                pl.BlockSpec((1, C), lambda i: (0, 0)),
            ],
            out_specs=[
                pl.BlockSpec((1, V, V), lambda i: (i, 0, 0)),
                pl.BlockSpec((1, 1, V), lambda i: (i, 0, 0)),
                pl.BlockSpec((1, 1, V), lambda i: (i, 0, 0)),
            ],
            out_shape=[
                jax.ShapeDtypeStruct((HALF, V, V), jnp.float32),
                jax.ShapeDtypeStruct((HALF, 1, V), jnp.float32),
                jax.ShapeDtypeStruct((HALF, 1, V), jnp.float32),
            ],
        )(x_half, W, b2, a1, a2)
        s2 = sc_topk(p.reshape(rows, V), t2.reshape(rows))
        return s2, t1

    s2s, t1s = [], []
    for i in range(NSPLIT):
        s2_i, t1_i = half(x[i * HALF:(i + 1) * HALF])
        s2s.append(s2_i.reshape(HALF, V * K))
        t1s.append(t1_i.reshape(HALF, V))
    h = pl.pallas_call(
        _softmax0_kernel,
        out_shape=jax.ShapeDtypeStruct((BATCH, V * K), jnp.float32),
    )(*s2s, *t1s)
    return h.reshape(BATCH, V, K)


# speedup vs baseline: 1.4048x; 1.0056x over previous
import jax
import jax.numpy as jnp
from jax import lax
from jax.experimental import pallas as pl
from jax.experimental.pallas import tpu as pltpu
from jax.experimental.pallas import tpu_sc as plsc

BATCH = 64
CIN = 128
C = 256
V = 256
K = 16

NSPLIT = 2
HALF = BATCH // NSPLIT
NSUB = 32
SLAB = 128


def _pairwise_kernel(x_ref, w_ref, b_ref, a1_ref, a2_ref,
                     p_ref, t1_ref, t2_ref):
    x_b = x_ref[0]
    pre = jnp.dot(w_ref[...], x_b, preferred_element_type=jnp.float32)
    pre = pre + b_ref[...]
    g = jax.lax.dot_general(pre, pre, (((0,), (0,)), ((), ())),
                            preferred_element_type=jnp.float32)
    d = jnp.sum(pre * pre, axis=0, keepdims=True)
    inner = -2.0 * g
    p_ref[0] = (-d - inner) - jnp.transpose(d)
    t1_ref[0] = jnp.dot(a1_ref[...], pre, preferred_element_type=jnp.float32)
    t2_ref[0] = jnp.dot(a2_ref[...], pre, preferred_element_type=jnp.float32)


def _merge16(ak, av, bk, bv):
    rbk = lax.rev(bk, (0,))
    rbv = lax.rev(bv, (0,))
    cm = ak >= rbk
    mk = jnp.where(cm, ak, rbk)
    mv = jnp.where(cm, av, rbv)
    return plsc.sort_key_val(mk, mv, descending=True)


def _make_sc_topk(rows):
    rpw = rows // NSUB
    nslab = rpw // SLAB

    def _sc_topk_body(p_hbm, t2_hbm, out_hbm, slab_buf, t2_buf, out_buf):
        wid = lax.axis_index("s") * 2 + lax.axis_index("c")
        base_row = wid * rpw
        pltpu.sync_copy(t2_hbm.at[pl.ds(base_row, rpw)], t2_buf)
        iota16 = lax.broadcasted_iota(jnp.int32, (16,), 0)
        for slab in range(nslab):
            row0 = base_row + slab * SLAB
            pltpu.sync_copy(p_hbm.at[pl.ds(row0, SLAB)], slab_buf)

            @plsc.parallel_loop(0, SLAB, unroll=2)
            def row_body(r, slab=slab):
                lists = []
                for c in range(V // 16):
                    vals = slab_buf[r, pl.ds(16 * c, 16)]
                    kk, vv = plsc.sort_key_val(vals, iota16 + 16 * c,
                                               descending=True)
                    lists.append((kk, vv))
                while len(lists) > 1:
                    nxt = []
                    for i in range(0, len(lists), 2):
                        ak, av = lists[i]
                        bk, bv = lists[i + 1]
                        nxt.append(_merge16(ak, av, bk, bv))
                    lists = nxt
                top_idx = lists[0][1]
                rsub = slab * SLAB + r
                adj = top_idx + jnp.bitwise_and(rsub, 256)
                tvals = plsc.load_gather(t2_buf, [adj])
                out_buf[r, :] = tvals

            pltpu.sync_copy(out_buf, out_hbm.at[pl.ds(row0, SLAB)])

    return pl.kernel(
        _sc_topk_body,
        out_type=jax.ShapeDtypeStruct((rows, K), jnp.float32),
        mesh=plsc.VectorSubcoreMesh(core_axis_name="c", subcore_axis_name="s"),
        scratch_types=[
            pltpu.VMEM((SLAB, V), jnp.float32),
            pltpu.VMEM((rpw,), jnp.float32),
            pltpu.VMEM((SLAB, K), jnp.float32),
        ],
        compiler_params=pltpu.CompilerParams(needs_layout_passes=False),
    )


def _softmax0_kernel(*refs):
    s_refs = refs[:NSPLIT]
    t_refs = refs[NSPLIT:2 * NSPLIT]
    out_ref = refs[-1]
    parts = []
    for s_ref, t_ref in zip(s_refs, t_refs):
        parts.append(s_ref[...] + jnp.concatenate([t_ref[...]] * K, axis=1))
    s = jnp.concatenate(parts, axis=0)
    m = jnp.max(s, axis=0, keepdims=True)
    e = jnp.exp(s - m)
    out_ref[...] = e / jnp.sum(e, axis=0, keepdims=True)


def kernel(x, W, b, a):
    a1 = a[:C, 0].reshape(1, C)
    a2 = a[C:, 0].reshape(1, C)
    b2 = b.reshape(C, 1)
    sc_topk = _make_sc_topk(HALF * V)

    def half(x_half):
        rows = HALF * V
        p, t1, t2 = pl.pallas_call(
            _pairwise_kernel,
            grid=(HALF,),
            in_specs=[
                pl.BlockSpec((1, CIN, V), lambda i: (i, 0, 0)),
                pl.BlockSpec((C, CIN), lambda i: (0, 0)),
                pl.BlockSpec((C, 1), lambda i: (0, 0)),
                pl.BlockSpec((1, C), lambda i: (0, 0)),
                pl.BlockSpec((1, C), lambda i: (0, 0)),
            ],
            out_specs=[
                pl.BlockSpec((1, V, V), lambda i: (i, 0, 0)),
                pl.BlockSpec((1, 1, V), lambda i: (i, 0, 0)),
                pl.BlockSpec((1, 1, V), lambda i: (i, 0, 0)),
            ],
            out_shape=[
                jax.ShapeDtypeStruct((HALF, V, V), jnp.float32),
                jax.ShapeDtypeStruct((HALF, 1, V), jnp.float32),
                jax.ShapeDtypeStruct((HALF, 1, V), jnp.float32),
            ],
        )(x_half, W, b2, a1, a2)
        s2 = sc_topk(p.reshape(rows, V), t2.reshape(rows))
        return s2, t1

    s2s, t1s = [], []
    for i in range(NSPLIT):
        s2_i, t1_i = half(x[i * HALF:(i + 1) * HALF])
        s2s.append(s2_i.reshape(HALF, V * K))
        t1s.append(t1_i.reshape(HALF, V))
    h = pl.pallas_call(
        _softmax0_kernel,
        out_shape=jax.ShapeDtypeStruct((BATCH, V * K), jnp.float32),
    )(*s2s, *t1s)
    return h.reshape(BATCH, V, K)
